# baseline (speedup 1.0000x reference)
import jax
import jax.numpy as jnp
from jax.experimental import pallas as pl
from jax.experimental.pallas import tpu as pltpu

CHUNKS = 5


def _leaky(v):
    return jnp.where(v >= 0, v, 0.01 * v)


def _tr(a, b):
    return jax.lax.dot_general(a, b, (((1,), (1,)), ((), ())),
                               preferred_element_type=jnp.float32)


def _make_kernel(n, c, h):
    def body(x_ref, adj_ref, w1_ref, b1_ref, w2_ref, b2_ref,
             wrel_ref, brel_ref, wroot_ref, w3_ref, b3_ref,
             w4_ref, b4_ref, out_ref, h2_scr, agg_scr, cnt_scr):
        i = pl.program_id(0)

        x_c = x_ref[0, 0]
        h1 = _leaky(_tr(x_c, w1_ref[...]) + b1_ref[...])
        h2 = _leaky(_tr(h1, w2_ref[...]) + b2_ref[...])
        h2_scr[pl.ds(i * c, c), :] = h2

        a = (adj_ref[0, 0] != 0).astype(jnp.float32)
        p_agg = jax.lax.dot_general(a, h2, (((0,), (0,)), ((), ())),
                                    preferred_element_type=jnp.float32)
        p_cnt = jax.lax.dot_general(a, jnp.ones((c, h), jnp.float32),
                                    (((0,), (0,)), ((), ())),
                                    preferred_element_type=jnp.float32)

        @pl.when(i == 0)
        def _init():
            agg_scr[...] = p_agg
            cnt_scr[...] = p_cnt

        @pl.when(i > 0)
        def _accum():
            agg_scr[...] += p_agg
            cnt_scr[...] += p_cnt

        @pl.when(i == CHUNKS - 1)
        def _epilogue():
            mean = agg_scr[...] / jnp.maximum(cnt_scr[...], 1.0)
            h2f = h2_scr[...]
            conv = _leaky(_tr(mean, wrel_ref[...]) + brel_ref[...]
                          + _tr(h2f, wroot_ref[...]))
            h3 = _leaky(_tr(conv, w3_ref[...]) + b3_ref[...])
            out_ref[...] = (jnp.sum(h3 * w4_ref[...], axis=1, keepdims=True)
                            + b4_ref[...])

    return body


def kernel(x, edge_indexs, edgenum, W_fc1, b_fc1, W_fc2, b_fc2, W_rel, b_rel,
           W_root, W_fc3, b_fc3, W_fc4, b_fc4):
    batch, seq_len, n, f_in = x.shape
    h = W_fc1.shape[0]
    last = seq_len - 1
    c = n // CHUNKS

    def full(shape):
        return pl.BlockSpec(shape, lambda i: tuple(0 for _ in shape))

    y = pl.pallas_call(
        _make_kernel(n, c, h),
        out_shape=jax.ShapeDtypeStruct((n, 1), jnp.float32),
        grid=(CHUNKS,),
        in_specs=[
            pl.BlockSpec((1, 1, c, f_in), lambda i: (0, last, i, 0)),
            pl.BlockSpec((1, 1, c, n), lambda i: (0, last, i, 0)),
            full((h, f_in)), full((1, h)),
            full((h, h)), full((1, h)),
            full((h, h)), full((1, h)),
            full((h, h)),
            full((h, h)), full((1, h)),
            full((1, h)), full((1, 1)),
        ],
        out_specs=pl.BlockSpec((n, 1), lambda i: (0, 0)),
        scratch_shapes=[
            pltpu.VMEM((n, h), jnp.float32),
            pltpu.VMEM((n, h), jnp.float32),
            pltpu.VMEM((n, h), jnp.float32),
        ],
    )(x, edge_indexs,
      W_fc1, b_fc1[None, :], W_fc2, b_fc2[None, :],
      W_rel, b_rel[None, :], W_root,
      W_fc3, b_fc3[None, :],
      W_fc4, b_fc4[None, :])

    return y.reshape(batch, n, 1)


# speedup vs baseline: 1.2814x; 1.2814x over previous
import jax
import jax.numpy as jnp
from jax.experimental import pallas as pl


def _leaky(v):
    return jnp.where(v >= 0, v, 0.01 * v)


def _tr(a, b):
    return jax.lax.dot_general(a, b, (((1,), (1,)), ((), ())),
                               preferred_element_type=jnp.float32)


def _gnn_last_step_kernel(x_ref, adj_ref, w1_ref, b1_ref, w2_ref, b2_ref,
                          wrel_ref, brel_ref, wroot_ref, w3_ref, b3_ref,
                          w4_ref, b4_ref, out_ref):
    x = x_ref[0, 0]
    h1 = _leaky(_tr(x, w1_ref[...]) + b1_ref[...])
    h2 = _leaky(_tr(h1, w2_ref[...]) + b2_ref[...])

    a = (adj_ref[0, 0] != 0).astype(jnp.float32)
    n, h = h2.shape[0], h2.shape[1]
    agg = jax.lax.dot_general(a, h2, (((0,), (0,)), ((), ())),
                              preferred_element_type=jnp.float32)
    cnt = jax.lax.dot_general(a, jnp.ones((n, h), jnp.float32),
                              (((0,), (0,)), ((), ())),
                              preferred_element_type=jnp.float32)
    mean = agg / jnp.maximum(cnt, 1.0)

    conv = _leaky(_tr(mean, wrel_ref[...]) + brel_ref[...]
                  + _tr(h2, wroot_ref[...]))
    h3 = _leaky(_tr(conv, w3_ref[...]) + b3_ref[...])
    y = jnp.sum(h3 * w4_ref[...], axis=1, keepdims=True) + b4_ref[...]
    out_ref[...] = y


def kernel(x, edge_indexs, edgenum, W_fc1, b_fc1, W_fc2, b_fc2, W_rel, b_rel,
           W_root, W_fc3, b_fc3, W_fc4, b_fc4):
    batch, seq_len, n, f_in = x.shape
    h = W_fc1.shape[0]
    last = seq_len - 1

    def full(shape):
        return pl.BlockSpec(shape, lambda i: tuple(0 for _ in shape))

    y = pl.pallas_call(
        _gnn_last_step_kernel,
        out_shape=jax.ShapeDtypeStruct((n, 1), jnp.float32),
        grid=(1,),
        in_specs=[
            pl.BlockSpec((1, 1, n, f_in), lambda i: (0, last, 0, 0)),
            pl.BlockSpec((1, 1, n, n), lambda i: (0, last, 0, 0)),
            full((h, f_in)), full((1, h)),
            full((h, h)), full((1, h)),
            full((h, h)), full((1, h)),
            full((h, h)),
            full((h, h)), full((1, h)),
            full((1, h)), full((1, 1)),
        ],
        out_specs=pl.BlockSpec((n, 1), lambda i: (0, 0)),
    )(x, edge_indexs,
      W_fc1, b_fc1[None, :], W_fc2, b_fc2[None, :],
      W_rel, b_rel[None, :], W_root,
      W_fc3, b_fc3[None, :],
      W_fc4, b_fc4[None, :])

    return y.reshape(batch, n, 1)
